# Initial kernel scaffold; baseline (speedup 1.0000x reference)
import jax
import jax.numpy as jnp
from jax.experimental import pallas as pl


def kernel(h, W, b):
    raise NotImplementedError("write your pallas kernel here")


# speedup vs baseline: 1.6204x; 1.6204x over previous
import functools

import jax
import jax.numpy as jnp
from jax import lax
from jax.experimental import pallas as pl
from jax.experimental.pallas import tpu as pltpu
from jax.experimental.pallas import tpu_sc as plsc

B = 4
N = 100000
F = 128
K = 50000
NPAD = 131072
LANES = 128
BLANES = 32
ROWS = NPAD // BLANES
SCORE_CHUNK = 4000


def _score_body(h_ref, w_ref, b_ref, s_ref, h2_ref):
    hb = h_ref[0]
    acc = lax.dot_general(hb.astype(jnp.bfloat16), w_ref[...].astype(jnp.bfloat16),
                          (((1,), (0,)), ((), ())),
                          preferred_element_type=jnp.float32)
    s = jax.nn.sigmoid(acc[:, 0] + b_ref[0])
    s_ref[0, 0] = s
    h2_ref[0] = hb * s[:, None]


def _scores_and_h2(h, W, b):
    nchunk = N // SCORE_CHUNK
    w_pad = jnp.zeros((F, 128), jnp.float32).at[:, 0].set(W[0])
    s, h2 = pl.pallas_call(
        _score_body,
        grid=(B, nchunk),
        in_specs=[
            pl.BlockSpec((1, SCORE_CHUNK, F), lambda i, j: (i, j, 0)),
            pl.BlockSpec((F, 128), lambda i, j: (0, 0)),
            pl.BlockSpec(memory_space=pltpu.SMEM),
        ],
        out_specs=[
            pl.BlockSpec((1, 1, SCORE_CHUNK), lambda i, j: (i * nchunk + j, 0, 0)),
            pl.BlockSpec((1, SCORE_CHUNK, F), lambda i, j: (i, j, 0)),
        ],
        out_shape=[
            jax.ShapeDtypeStruct((B * nchunk, 1, SCORE_CHUNK), jnp.float32),
            jax.ShapeDtypeStruct((B, N, F), jnp.float32),
        ],
    )(h, w_pad, b)
    return s.reshape(B, N), h2


def _partner(x, bit_hi, dist, size, axis):
    up = pltpu.roll(x, size - dist, axis=axis)
    dn = pltpu.roll(x, dist, axis=axis)
    return jnp.where(bit_hi, dn, up)


def _substep(key, idx, r_iota, c_iota, e, ek, axis, dist, size, active):
    if axis == 0:
        bit = (r_iota >> (e - 5)) & 1
    else:
        bit = (c_iota >> e) & 1
    bit_hi = bit == 1
    kp = _partner(key, bit_hi, dist, size, axis)
    ip = _partner(idx, bit_hi, dist, size, axis)
    bk_lane = (c_iota >> jnp.minimum(ek, 4)) & 1
    bk_row = (r_iota >> jnp.maximum(ek - 5, 0)) & 1
    desc = jnp.where(ek <= 4, bk_lane, bk_row) == 0
    f = (key > kp) | ((key == kp) & (idx < ip))
    lower = ~bit_hi
    keep = (f == (lower == desc)) | jnp.logical_not(active)
    return jnp.where(keep, key, kp), jnp.where(keep, idx, ip)


def _sort_body(k_ref, ks_ref, is_ref):
    key = k_ref[...]
    r_iota = lax.broadcasted_iota(jnp.int32, (ROWS, LANES), 0)
    c_iota = lax.broadcasted_iota(jnp.int32, (ROWS, LANES), 1) & (BLANES - 1)
    idx = r_iota * BLANES + c_iota

    def phase(ek, carry):
        key, idx = carry

        def rowstep(t, c):
            key, idx = c
            e = ek - 1 - t
            d = 1 << (e - 5)
            return _substep(key, idx, r_iota, c_iota, e, ek, 0, d, ROWS,
                            jnp.bool_(True))

        key, idx = lax.fori_loop(0, jnp.maximum(ek - 5, 0), rowstep,
                                 (key, idx))

        def lanestep(t, c):
            key, idx = c
            e = 4 - t
            d = 1 << e
            return _substep(key, idx, r_iota, c_iota, e, ek, 1, d, LANES,
                            e < ek)

        return lax.fori_loop(0, 5, lanestep, (key, idx))

    key, idx = lax.fori_loop(1, 18, phase, (key, idx))
    ks_ref[...] = key
    is_ref[...] = idx


def _sort_grid(keys_grid):
    return pl.pallas_call(
        _sort_body,
        out_shape=[
            jax.ShapeDtypeStruct((ROWS, LANES), jnp.float32),
            jax.ShapeDtypeStruct((ROWS, LANES), jnp.int32),
        ],
    )(keys_grid)


_WPB = 8
_WIN = 400
_BIG = 6400
_KPAD = 50048


def _gather_topk(h2, idx):
    mesh = plsc.VectorSubcoreMesh(core_axis_name="c", subcore_axis_name="s")
    idx_flat = jnp.pad(idx, ((0, 0), (0, _KPAD - K))).reshape(B * _KPAD)

    @functools.partial(
        pl.kernel,
        mesh=mesh,
        out_type=jax.ShapeDtypeStruct((B, K, F), jnp.float32),
        scratch_types=[
            pltpu.VMEM((_WIN,), jnp.int32),
            pltpu.VMEM((_WIN, F), jnp.float32),
            pltpu.SemaphoreType.DMA,
        ],
    )
    def gather_k(h2_hbm, idx_hbm, out_hbm, idx_v, rows_v, sem):
        wid = lax.axis_index("s") * 2 + lax.axis_index("c")
        bat = wid // _WPB
        wb = wid % _WPB
        base = wb * _BIG
        nwin = jnp.where(wb == _WPB - 1, (K - (_WPB - 1) * _BIG) // _WIN,
                         _BIG // _WIN)

        def win(i, carry):
            off = base + i * _WIN
            pltpu.sync_copy(idx_hbm.at[pl.ds(bat * _KPAD + off, _WIN)], idx_v)
            pltpu.async_copy(h2_hbm.at[bat].at[idx_v], rows_v, sem).wait()
            pltpu.sync_copy(rows_v, out_hbm.at[bat, pl.ds(off, _WIN)])
            return carry

        lax.fori_loop(0, nwin, win, 0)

    return gather_k(h2, idx_flat)


def kernel(h, W, b):
    s, h2 = _scores_and_h2(h, W, b)
    s_pad = jnp.pad(s, ((0, 0), (0, NPAD - N)), constant_values=-1.0)
    grid = s_pad.reshape(B, ROWS, BLANES).transpose(1, 0, 2).reshape(ROWS, LANES)
    _, idx_grid = _sort_grid(grid)
    idx_sorted = (idx_grid.reshape(ROWS, B, BLANES)
                  .transpose(1, 0, 2).reshape(B, NPAD)[:, :K])
    return _gather_topk(h2, idx_sorted)


# speedup vs baseline: 1.7581x; 1.0849x over previous
import functools

import jax
import jax.numpy as jnp
from jax import lax
from jax.experimental import pallas as pl
from jax.experimental.pallas import tpu as pltpu
from jax.experimental.pallas import tpu_sc as plsc

B = 4
N = 100000
F = 128
K = 50000
NPAD = 131072
LANES = 128
BLANES = 32
ROWS = NPAD // BLANES
SCORE_CHUNK = 4000


def _score_body(h_ref, w_ref, b_ref, s_ref, h2_ref):
    hb = h_ref[0]
    acc = lax.dot_general(hb.astype(jnp.bfloat16), w_ref[...].astype(jnp.bfloat16),
                          (((1,), (0,)), ((), ())),
                          preferred_element_type=jnp.float32)
    s = jax.nn.sigmoid(acc[:, 0] + b_ref[0])
    s_ref[0, 0] = s
    h2_ref[0] = hb * s[:, None]


def _scores_and_h2(h, W, b):
    nchunk = N // SCORE_CHUNK
    w_pad = jnp.zeros((F, 128), jnp.float32).at[:, 0].set(W[0])
    s, h2 = pl.pallas_call(
        _score_body,
        grid=(B, nchunk),
        in_specs=[
            pl.BlockSpec((1, SCORE_CHUNK, F), lambda i, j: (i, j, 0)),
            pl.BlockSpec((F, 128), lambda i, j: (0, 0)),
            pl.BlockSpec(memory_space=pltpu.SMEM),
        ],
        out_specs=[
            pl.BlockSpec((1, 1, SCORE_CHUNK), lambda i, j: (i * nchunk + j, 0, 0)),
            pl.BlockSpec((1, SCORE_CHUNK, F), lambda i, j: (i, j, 0)),
        ],
        out_shape=[
            jax.ShapeDtypeStruct((B * nchunk, 1, SCORE_CHUNK), jnp.float32),
            jax.ShapeDtypeStruct((B, N, F), jnp.float32),
        ],
    )(h, w_pad, b)
    return s.reshape(B, N), h2


def _partner(x, bit_hi, dist, size, axis):
    up = pltpu.roll(x, size - dist, axis=axis)
    dn = pltpu.roll(x, dist, axis=axis)
    return jnp.where(bit_hi, dn, up)


def _substep(key, idx, r_iota, c_iota, desc, e, axis, dist, size, active):
    if axis == 0:
        bit = (r_iota >> (e - 5)) & 1
    else:
        bit = (c_iota >> e) & 1
    bit_hi = bit == 1
    kp = _partner(key, bit_hi, dist, size, axis)
    ip = _partner(idx, bit_hi, dist, size, axis)
    f = (key > kp) | ((key == kp) & (idx < ip))
    lower = ~bit_hi
    keep = (f == (lower == desc)) | jnp.logical_not(active)
    return jnp.where(keep, key, kp), jnp.where(keep, idx, ip)


def _run_phase(key, idx, r_iota, c_iota, ek, rows):
    bk_lane = (c_iota >> jnp.minimum(ek, 4)) & 1
    bk_row = (r_iota >> jnp.maximum(ek - 5, 0)) & 1
    desc = jnp.where(ek <= 4, bk_lane, bk_row) == 0

    def rowstep(t, c):
        key, idx = c
        e = ek - 1 - t
        d = 1 << (e - 5)
        return _substep(key, idx, r_iota, c_iota, desc, e, 0, d, rows,
                        jnp.bool_(True))

    key, idx = lax.fori_loop(0, jnp.maximum(ek - 5, 0), rowstep, (key, idx))

    def lanestep(t, c):
        key, idx = c
        e = 4 - t
        d = 1 << e
        return _substep(key, idx, r_iota, c_iota, desc, e, 1, d, LANES,
                        e < ek)

    return lax.fori_loop(0, 5, lanestep, (key, idx))


def _sort_body(k_ref, ks_ref, is_ref):
    key = k_ref[...]
    r_iota = lax.broadcasted_iota(jnp.int32, (ROWS, LANES), 0)
    c_iota = lax.broadcasted_iota(jnp.int32, (ROWS, LANES), 1) & (BLANES - 1)
    idx = r_iota * BLANES + c_iota

    def phase(ek, carry):
        key, idx = carry
        return _run_phase(key, idx, r_iota, c_iota, ek, ROWS)

    key, idx = lax.fori_loop(1, 17, phase, (key, idx))
    key, idx = _substep(key, idx, r_iota, c_iota,
                        jnp.bool_(True), 16, 0, ROWS // 2, ROWS,
                        jnp.bool_(True))
    HR = ROWS // 2
    kh, ih = key[:HR], idx[:HR]
    rh, ch = r_iota[:HR], c_iota[:HR]

    def phase17(t, carry):
        kh, ih = carry
        e = 15 - t
        d = 1 << (e - 5)
        return _substep(kh, ih, rh, ch, jnp.bool_(True), e, 0, d, HR,
                        jnp.bool_(True))

    kh, ih = lax.fori_loop(0, 11, phase17, (kh, ih))

    def lane17(t, carry):
        kh, ih = carry
        e = 4 - t
        return _substep(kh, ih, rh, ch, jnp.bool_(True), e, 1, 1 << e, LANES,
                        jnp.bool_(True))

    kh, ih = lax.fori_loop(0, 5, lane17, (kh, ih))
    ks_ref[...] = kh
    is_ref[...] = ih


def _sort_grid(keys_grid):
    return pl.pallas_call(
        _sort_body,
        out_shape=[
            jax.ShapeDtypeStruct((ROWS // 2, LANES), jnp.float32),
            jax.ShapeDtypeStruct((ROWS // 2, LANES), jnp.int32),
        ],
    )(keys_grid)


_WPB = 8
_WIN = 400
_BIG = 6400
_KPAD = 50048


def _gather_topk(h2, idx):
    mesh = plsc.VectorSubcoreMesh(core_axis_name="c", subcore_axis_name="s")
    idx_flat = jnp.pad(idx, ((0, 0), (0, _KPAD - K))).reshape(B * _KPAD)

    @functools.partial(
        pl.kernel,
        mesh=mesh,
        out_type=jax.ShapeDtypeStruct((B, K, F), jnp.float32),
        scratch_types=[
            pltpu.VMEM((_WIN,), jnp.int32),
            pltpu.VMEM((_WIN, F), jnp.float32),
            pltpu.SemaphoreType.DMA,
        ],
    )
    def gather_k(h2_hbm, idx_hbm, out_hbm, idx_v, rows_v, sem):
        wid = lax.axis_index("s") * 2 + lax.axis_index("c")
        bat = wid // _WPB
        wb = wid % _WPB
        base = wb * _BIG
        nwin = jnp.where(wb == _WPB - 1, (K - (_WPB - 1) * _BIG) // _WIN,
                         _BIG // _WIN)

        def win(i, carry):
            off = base + i * _WIN
            pltpu.sync_copy(idx_hbm.at[pl.ds(bat * _KPAD + off, _WIN)], idx_v)
            pltpu.async_copy(h2_hbm.at[bat].at[idx_v], rows_v, sem).wait()
            pltpu.sync_copy(rows_v, out_hbm.at[bat, pl.ds(off, _WIN)])
            return carry

        lax.fori_loop(0, nwin, win, 0)

    return gather_k(h2, idx_flat)


def kernel(h, W, b):
    s, h2 = _scores_and_h2(h, W, b)
    s_pad = jnp.pad(s, ((0, 0), (0, NPAD - N)), constant_values=-1.0)
    grid = s_pad.reshape(B, ROWS, BLANES).transpose(1, 0, 2).reshape(ROWS, LANES)
    _, idx_grid = _sort_grid(grid)
    idx_sorted = (idx_grid.reshape(ROWS // 2, B, BLANES)
                  .transpose(1, 0, 2).reshape(B, NPAD // 2)[:, :K])
    return _gather_topk(h2, idx_sorted)


# speedup vs baseline: 1.7821x; 1.0137x over previous
import functools

import jax
import jax.numpy as jnp
from jax import lax
from jax.experimental import pallas as pl
from jax.experimental.pallas import tpu as pltpu
from jax.experimental.pallas import tpu_sc as plsc

B = 4
N = 100000
F = 128
K = 50000
NPAD = 131072
LANES = 128
BLANES = 32
ROWS = NPAD // BLANES
SCORE_CHUNK = 4000


def _score_body(h_ref, w_ref, b_ref, s_ref, h2_ref):
    hb = h_ref[0]
    acc = lax.dot_general(hb.astype(jnp.bfloat16), w_ref[...].astype(jnp.bfloat16),
                          (((1,), (0,)), ((), ())),
                          preferred_element_type=jnp.float32)
    s = jax.nn.sigmoid(acc[:, 0] + b_ref[0])
    s_ref[0, 0] = s
    h2_ref[0] = hb * s[:, None]


def _scores_and_h2(h, W, b):
    nchunk = N // SCORE_CHUNK
    w_pad = jnp.zeros((F, 128), jnp.float32).at[:, 0].set(W[0])
    s, h2 = pl.pallas_call(
        _score_body,
        grid=(B, nchunk),
        in_specs=[
            pl.BlockSpec((1, SCORE_CHUNK, F), lambda i, j: (i, j, 0)),
            pl.BlockSpec((F, 128), lambda i, j: (0, 0)),
            pl.BlockSpec(memory_space=pltpu.SMEM),
        ],
        out_specs=[
            pl.BlockSpec((1, 1, SCORE_CHUNK), lambda i, j: (i * nchunk + j, 0, 0)),
            pl.BlockSpec((1, SCORE_CHUNK, F), lambda i, j: (i, j, 0)),
        ],
        out_shape=[
            jax.ShapeDtypeStruct((B * nchunk, 1, SCORE_CHUNK), jnp.float32),
            jax.ShapeDtypeStruct((B, N, F), jnp.float32),
        ],
    )(h, w_pad, b)
    return s.reshape(B, N), h2


def _partner(x, bit_hi, dist, size, axis):
    up = pltpu.roll(x, size - dist, axis=axis)
    dn = pltpu.roll(x, dist, axis=axis)
    return jnp.where(bit_hi, dn, up)


def _substep(key, idx, r_iota, c_iota, desc, e, axis, dist, size):
    if axis == 0:
        bit = (r_iota >> (e - 5)) & 1
    else:
        bit = (c_iota >> e) & 1
    bit_hi = bit == 1
    kp = _partner(key, bit_hi, dist, size, axis)
    ip = _partner(idx, bit_hi, dist, size, axis)
    f = (key > kp) | ((key == kp) & (idx < ip))
    lower = ~bit_hi
    keep = f == (lower == desc)
    return jnp.where(keep, key, kp), jnp.where(keep, idx, ip)


def _run_phase(key, idx, r_iota, c_iota, ek, rows):
    bk_lane = (c_iota >> jnp.minimum(ek, 4)) & 1
    bk_row = (r_iota >> jnp.maximum(ek - 5, 0)) & 1
    desc = jnp.where(ek <= 4, bk_lane, bk_row) == 0

    def rowstep(t, c):
        key, idx = c
        e = ek - 1 - t
        d = 1 << (e - 5)
        return _substep(key, idx, r_iota, c_iota, desc, e, 0, d, rows)

    key, idx = lax.fori_loop(0, jnp.maximum(ek - 5, 0), rowstep, (key, idx))

    def lanestep(t, c):
        key, idx = c
        e = 4 - t
        d = 1 << e
        return _substep(key, idx, r_iota, c_iota, desc, e, 1, d, LANES)

    return lax.fori_loop(5 - jnp.minimum(ek, 5), 5, lanestep, (key, idx))


def _sort_body(k_ref, ks_ref, is_ref):
    key = k_ref[...]
    r_iota = lax.broadcasted_iota(jnp.int32, (ROWS, LANES), 0)
    c_iota = lax.broadcasted_iota(jnp.int32, (ROWS, LANES), 1) & (BLANES - 1)
    idx = r_iota * BLANES + c_iota

    def phase(ek, carry):
        key, idx = carry
        return _run_phase(key, idx, r_iota, c_iota, ek, ROWS)

    key, idx = lax.fori_loop(1, 17, phase, (key, idx))
    key, idx = _substep(key, idx, r_iota, c_iota,
                        jnp.bool_(True), 16, 0, ROWS // 2, ROWS)
    HR = ROWS // 2
    kh, ih = key[:HR], idx[:HR]
    rh, ch = r_iota[:HR], c_iota[:HR]

    def phase17(t, carry):
        kh, ih = carry
        e = 15 - t
        d = 1 << (e - 5)
        return _substep(kh, ih, rh, ch, jnp.bool_(True), e, 0, d, HR)

    kh, ih = lax.fori_loop(0, 11, phase17, (kh, ih))

    def lane17(t, carry):
        kh, ih = carry
        e = 4 - t
        return _substep(kh, ih, rh, ch, jnp.bool_(True), e, 1, 1 << e, LANES)

    kh, ih = lax.fori_loop(0, 5, lane17, (kh, ih))
    ks_ref[...] = kh
    is_ref[...] = ih


def _sort_grid(keys_grid):
    return pl.pallas_call(
        _sort_body,
        out_shape=[
            jax.ShapeDtypeStruct((ROWS // 2, LANES), jnp.float32),
            jax.ShapeDtypeStruct((ROWS // 2, LANES), jnp.int32),
        ],
    )(keys_grid)


_WPB = 8
_WIN = 400
_BIG = 6400
_KPAD = 50048


def _gather_topk(h2, idx):
    mesh = plsc.VectorSubcoreMesh(core_axis_name="c", subcore_axis_name="s")
    idx_flat = jnp.pad(idx, ((0, 0), (0, _KPAD - K))).reshape(B * _KPAD)

    @functools.partial(
        pl.kernel,
        mesh=mesh,
        out_type=jax.ShapeDtypeStruct((B, K, F), jnp.float32),
        scratch_types=[
            pltpu.VMEM((_WIN,), jnp.int32),
            pltpu.VMEM((_WIN, F), jnp.float32),
            pltpu.SemaphoreType.DMA,
        ],
    )
    def gather_k(h2_hbm, idx_hbm, out_hbm, idx_v, rows_v, sem):
        wid = lax.axis_index("s") * 2 + lax.axis_index("c")
        bat = wid // _WPB
        wb = wid % _WPB
        base = wb * _BIG
        nwin = jnp.where(wb == _WPB - 1, (K - (_WPB - 1) * _BIG) // _WIN,
                         _BIG // _WIN)

        def win(i, carry):
            off = base + i * _WIN
            pltpu.sync_copy(idx_hbm.at[pl.ds(bat * _KPAD + off, _WIN)], idx_v)
            pltpu.async_copy(h2_hbm.at[bat].at[idx_v], rows_v, sem).wait()
            pltpu.sync_copy(rows_v, out_hbm.at[bat, pl.ds(off, _WIN)])
            return carry

        lax.fori_loop(0, nwin, win, 0)

    return gather_k(h2, idx_flat)


def kernel(h, W, b):
    s, h2 = _scores_and_h2(h, W, b)
    s_pad = jnp.pad(s, ((0, 0), (0, NPAD - N)), constant_values=-1.0)
    grid = s_pad.reshape(B, ROWS, BLANES).transpose(1, 0, 2).reshape(ROWS, LANES)
    _, idx_grid = _sort_grid(grid)
    idx_sorted = (idx_grid.reshape(ROWS // 2, B, BLANES)
                  .transpose(1, 0, 2).reshape(B, NPAD // 2)[:, :K])
    return _gather_topk(h2, idx_sorted)
